# baseline (speedup 1.0000x reference)
import functools

import jax
import jax.numpy as jnp
from jax import lax
from jax.experimental import pallas as pl
from jax.experimental.pallas import tpu as pltpu
from jax.experimental.pallas import tpu_sc as plsc

LANES = 16


def _make_sc_kernel(n_rows: int, row_words: int):
    info = plsc.get_sparse_core_info()
    nc, ns = info.num_cores, info.num_subcores
    nw = nc * ns
    rows_per_w = n_rows // nw
    chunk_rows = 16
    n_chunks = rows_per_w // chunk_rows
    chunk_words = chunk_rows * row_words
    n_vecs = row_words // LANES

    mesh = plsc.VectorSubcoreMesh(core_axis_name="c", subcore_axis_name="s")

    @functools.partial(
        pl.kernel,
        mesh=mesh,
        out_type=jax.ShapeDtypeStruct((n_rows * row_words,), jnp.float32),
        scratch_types=[
            pltpu.VMEM((row_words,), jnp.float32),
            pltpu.VMEM((chunk_words,), jnp.float32),
        ],
    )
    def k(x_hbm, tab_hbm, out_hbm, tab_v, buf):
        wid = lax.axis_index("s") * nc + lax.axis_index("c")
        base = wid * rows_per_w * row_words
        pltpu.sync_copy(tab_hbm, tab_v)

        def chunk_body(ci, carry):
            off = base + ci * chunk_words
            pltpu.sync_copy(x_hbm.at[pl.ds(off, chunk_words)], buf)

            def jbody(j, carry):
                tvec = tab_v[pl.ds(j * LANES, LANES)]

                def rbody(r, carry):
                    p = r * row_words + j * LANES
                    buf[pl.ds(p, LANES)] += tvec
                    return carry

                return lax.fori_loop(0, chunk_rows, rbody, carry)

            lax.fori_loop(0, n_vecs, jbody, 0)
            pltpu.sync_copy(buf, out_hbm.at[pl.ds(off, chunk_words)])
            return carry

        lax.fori_loop(0, n_chunks, chunk_body, 0)

    return k


def kernel(visual_embeddings, view_embed_weight):
    b, v, d = visual_embeddings.shape
    row_words = v * d
    x = visual_embeddings.reshape(b * row_words)
    tab = view_embed_weight.reshape(row_words)
    out = _make_sc_kernel(b, row_words)(x, tab)
    return out.reshape(b, v, d)


# speedup vs baseline: 1.0161x; 1.0161x over previous
import functools

import jax
import jax.numpy as jnp
from jax import lax
from jax.experimental import pallas as pl
from jax.experimental.pallas import tpu as pltpu
from jax.experimental.pallas import tpu_sc as plsc

LANES = 16
UNROLL = 4


def _make_sc_kernel(n_rows: int, row_words: int):
    info = plsc.get_sparse_core_info()
    nc, ns = info.num_cores, info.num_subcores
    nw = nc * ns
    rows_per_w = n_rows // nw
    chunk_rows = 16
    n_chunks = rows_per_w // chunk_rows
    chunk_words = chunk_rows * row_words
    n_jblk = row_words // (LANES * UNROLL)

    mesh = plsc.VectorSubcoreMesh(core_axis_name="c", subcore_axis_name="s")

    @functools.partial(
        pl.kernel,
        mesh=mesh,
        out_type=jax.ShapeDtypeStruct((n_rows * row_words,), jnp.float32),
        scratch_types=[
            pltpu.VMEM((row_words,), jnp.float32),
            pltpu.VMEM((chunk_words,), jnp.float32),
            pltpu.VMEM((chunk_words,), jnp.float32),
            pltpu.SemaphoreType.DMA,
            pltpu.SemaphoreType.DMA,
            pltpu.SemaphoreType.DMA,
            pltpu.SemaphoreType.DMA,
        ],
    )
    def k(x_hbm, tab_hbm, out_hbm, tab_v, buf0, buf1,
          in_sem0, in_sem1, out_sem0, out_sem1):
        bufs = (buf0, buf1)
        in_sems = (in_sem0, in_sem1)
        out_sems = (out_sem0, out_sem1)

        wid = lax.axis_index("s") * nc + lax.axis_index("c")
        base = wid * rows_per_w * row_words
        pltpu.sync_copy(tab_hbm, tab_v)

        def start_in(ci, b):
            pltpu.async_copy(
                x_hbm.at[pl.ds(base + ci * chunk_words, chunk_words)],
                bufs[b], in_sems[b])

        def wait_in(b):
            pltpu.make_async_copy(
                x_hbm.at[pl.ds(0, chunk_words)], bufs[b], in_sems[b]).wait()

        def start_out(ci, b):
            pltpu.async_copy(
                bufs[b],
                out_hbm.at[pl.ds(base + ci * chunk_words, chunk_words)],
                out_sems[b])

        def wait_out(b):
            pltpu.make_async_copy(
                bufs[b], out_hbm.at[pl.ds(0, chunk_words)], out_sems[b]).wait()

        def compute(buf):
            def jb_body(jb, carry):
                t = jb * (LANES * UNROLL)
                tvs = [tab_v[pl.ds(t + u * LANES, LANES)] for u in range(UNROLL)]

                def rbody(r, carry):
                    p = r * row_words + t
                    for u in range(UNROLL):
                        buf[pl.ds(p + u * LANES, LANES)] += tvs[u]
                    return carry

                return lax.fori_loop(0, chunk_rows, rbody, carry)

            lax.fori_loop(0, n_jblk, jb_body, 0)

        start_in(0, 0)
        for ci in range(n_chunks):
            b = ci % 2
            nb = (ci + 1) % 2
            if ci + 1 < n_chunks:
                if ci >= 1:
                    wait_out(nb)
                start_in(ci + 1, nb)
            wait_in(b)
            compute(bufs[b])
            start_out(ci, b)
        wait_out(0)
        wait_out(1)

    return k


def kernel(visual_embeddings, view_embed_weight):
    b, v, d = visual_embeddings.shape
    row_words = v * d
    x = visual_embeddings.reshape(b * row_words)
    tab = view_embed_weight.reshape(row_words)
    out = _make_sc_kernel(b, row_words)(x, tab)
    return out.reshape(b, v, d)


# speedup vs baseline: 1.6687x; 1.6423x over previous
import functools

import jax
import jax.numpy as jnp
from jax import lax
from jax.experimental import pallas as pl
from jax.experimental.pallas import tpu as pltpu
from jax.experimental.pallas import tpu_sc as plsc

LANES = 16
UNROLL = 4


def _make_sc_kernel(n_rows: int, row_words: int):
    info = plsc.get_sparse_core_info()
    nc, ns = info.num_cores, info.num_subcores
    nw = nc * ns
    rows_per_w = n_rows // nw
    chunk_rows = 16
    n_chunks = rows_per_w // chunk_rows
    chunk_words = chunk_rows * row_words
    n_jblk = row_words // (LANES * UNROLL)

    mesh = plsc.VectorSubcoreMesh(core_axis_name="c", subcore_axis_name="s")

    @functools.partial(
        pl.kernel,
        mesh=mesh,
        out_type=jax.ShapeDtypeStruct((n_rows * row_words,), jnp.float32),
        scratch_types=[
            pltpu.VMEM((row_words,), jnp.float32),
            pltpu.VMEM((chunk_words,), jnp.float32),
            pltpu.VMEM((chunk_words,), jnp.float32),
            pltpu.SemaphoreType.DMA,
            pltpu.SemaphoreType.DMA,
            pltpu.SemaphoreType.DMA,
            pltpu.SemaphoreType.DMA,
        ],
    )
    def k(x_hbm, tab_hbm, out_hbm, tab_v, buf0, buf1,
          in_sem0, in_sem1, out_sem0, out_sem1):
        bufs = (buf0, buf1)
        in_sems = (in_sem0, in_sem1)
        out_sems = (out_sem0, out_sem1)

        wid = lax.axis_index("s") * nc + lax.axis_index("c")
        base = wid * rows_per_w * row_words
        pltpu.sync_copy(tab_hbm, tab_v)

        def start_in(ci, b):
            pltpu.async_copy(
                x_hbm.at[pl.ds(base + ci * chunk_words, chunk_words)],
                bufs[b], in_sems[b])

        def wait_in(b):
            pltpu.make_async_copy(
                x_hbm.at[pl.ds(0, chunk_words)], bufs[b], in_sems[b]).wait()

        def start_out(ci, b):
            pltpu.async_copy(
                bufs[b],
                out_hbm.at[pl.ds(base + ci * chunk_words, chunk_words)],
                out_sems[b])

        def wait_out(b):
            pltpu.make_async_copy(
                bufs[b], out_hbm.at[pl.ds(0, chunk_words)], out_sems[b]).wait()

        def compute(buf):
            def jb_body(jb, carry):
                t = jb * (LANES * UNROLL)
                tvs = [tab_v[pl.ds(t + u * LANES, LANES)] for u in range(UNROLL)]

                @plsc.parallel_loop(0, chunk_rows, unroll=4)
                def rbody(r):
                    p = r * row_words + t
                    for u in range(UNROLL):
                        buf[pl.ds(p + u * LANES, LANES)] += tvs[u]

                return carry

            lax.fori_loop(0, n_jblk, jb_body, 0)

        start_in(0, 0)
        for ci in range(n_chunks):
            b = ci % 2
            nb = (ci + 1) % 2
            if ci + 1 < n_chunks:
                if ci >= 1:
                    wait_out(nb)
                start_in(ci + 1, nb)
            wait_in(b)
            compute(bufs[b])
            start_out(ci, b)
        wait_out(0)
        wait_out(1)

    return k


def kernel(visual_embeddings, view_embed_weight):
    b, v, d = visual_embeddings.shape
    row_words = v * d
    x = visual_embeddings.reshape(b * row_words)
    tab = view_embed_weight.reshape(row_words)
    out = _make_sc_kernel(b, row_words)(x, tab)
    return out.reshape(b, v, d)


# speedup vs baseline: 3.7305x; 2.2356x over previous
import functools

import jax
import jax.numpy as jnp
from jax import lax
from jax.experimental import pallas as pl
from jax.experimental.pallas import tpu as pltpu
from jax.experimental.pallas import tpu_sc as plsc

LANES = 16
UNROLL = 4


def _make_sc_kernel(n_rows: int, n_views: int, d_model: int):
    info = plsc.get_sparse_core_info()
    nc, ns = info.num_cores, info.num_subcores
    nw = nc * ns
    rows_per_w = n_rows // nw
    chunk_rows = 16
    n_chunks = rows_per_w // chunk_rows
    n_jblk = d_model // (LANES * UNROLL)

    mesh = plsc.VectorSubcoreMesh(core_axis_name="c", subcore_axis_name="s")

    @functools.partial(
        pl.kernel,
        mesh=mesh,
        out_type=jax.ShapeDtypeStruct((n_rows, n_views, d_model), jnp.float32),
        scratch_types=[
            pltpu.VMEM((n_views, d_model), jnp.float32),
            pltpu.VMEM((chunk_rows, n_views, d_model), jnp.float32),
        ],
        compiler_params=pltpu.CompilerParams(use_tc_tiling_on_sc=True),
    )
    def k(x_hbm, tab_hbm, out_hbm, tab_v, buf):
        wid = lax.axis_index("s") * nc + lax.axis_index("c")
        base = wid * rows_per_w
        pltpu.sync_copy(tab_hbm, tab_v)

        def chunk_body(ci, carry):
            r0 = base + ci * chunk_rows
            pltpu.sync_copy(x_hbm.at[pl.ds(r0, chunk_rows)], buf)

            for v in range(n_views):
                def jb_body(jb, c2):
                    t = jb * (LANES * UNROLL)
                    tvs = [tab_v[v, pl.ds(t + u * LANES, LANES)]
                           for u in range(UNROLL)]

                    @plsc.parallel_loop(0, chunk_rows, unroll=4)
                    def rbody(r):
                        for u in range(UNROLL):
                            buf[r, v, pl.ds(t + u * LANES, LANES)] += tvs[u]

                    return c2

                lax.fori_loop(0, n_jblk, jb_body, 0)

            pltpu.sync_copy(buf, out_hbm.at[pl.ds(r0, chunk_rows)])
            return carry

        lax.fori_loop(0, n_chunks, chunk_body, 0)

    return k


def kernel(visual_embeddings, view_embed_weight):
    b, v, d = visual_embeddings.shape
    out = _make_sc_kernel(b, v, d)(visual_embeddings, view_embed_weight)
    return out
